# Initial kernel scaffold; baseline (speedup 1.0000x reference)
import functools

import jax
import jax.numpy as jnp
from jax import lax
from jax.experimental import pallas as pl
from jax.experimental.pallas import tpu as pltpu
from jax.experimental.pallas import tpu_sc as plsc

VOCAB = 100000
N_HIDDEN = 128
B = 16384
F = 26
TOT = B * F
NW = 32
CHUNK = TOT // NW
LANES = 16

VB = 12800
TC_GRID = (VOCAB + VB - 1) // VB


def _project_body(e_ref, w_ref, b_ref, o_ref):
    e = e_ref[...]
    w = w_ref[...]
    o_ref[...] = (
        lax.dot_general(w, e, (((1,), (1,)), ((), ())),
                        preferred_element_type=jnp.float32)
        + b_ref[0, 0]
    )


def _project_table(emb, W, b2d):
    return pl.pallas_call(
        _project_body,
        grid=(TC_GRID,),
        in_specs=[
            pl.BlockSpec((VB, N_HIDDEN), lambda i: (i, 0)),
            pl.BlockSpec((1, N_HIDDEN), lambda i: (0, 0)),
            pl.BlockSpec((1, 1), lambda i: (0, 0)),
        ],
        out_specs=pl.BlockSpec((1, VB), lambda i: (0, i)),
        out_shape=jax.ShapeDtypeStruct((1, VOCAB), jnp.float32),
    )(emb, W, b2d)


_SC_MESH = plsc.VectorSubcoreMesh(core_axis_name="c", subcore_axis_name="s")


@functools.partial(
    pl.kernel,
    out_type=jax.ShapeDtypeStruct((TOT,), jnp.float32),
    mesh=_SC_MESH,
    scratch_types=[
        pltpu.VMEM((VOCAB,), jnp.float32),
        pltpu.VMEM((CHUNK,), jnp.int32),
        pltpu.VMEM((CHUNK,), jnp.float32),
    ],
)
def _sc_gather(v_hbm, idx_hbm, out_hbm, v_v, idx_v, out_v):
    wid = lax.axis_index("s") * 2 + lax.axis_index("c")
    base = wid * CHUNK
    pltpu.sync_copy(v_hbm, v_v)
    pltpu.sync_copy(idx_hbm.at[pl.ds(base, CHUNK)], idx_v)

    def body(i, carry):
        idx = idx_v[pl.ds(i * LANES, LANES)]
        out_v[pl.ds(i * LANES, LANES)] = plsc.load_gather(v_v, [idx])
        return carry

    lax.fori_loop(0, CHUNK // LANES, body, 0)
    pltpu.sync_copy(out_v, out_hbm.at[pl.ds(base, CHUNK)])


def kernel(x, emb, W, b):
    v = _project_table(emb, W, b.reshape(1, 1)).reshape(VOCAB)
    idx = x.reshape(TOT).astype(jnp.int32)
    out = _sc_gather(v, idx)
    return out.reshape(B, F, 1)


# speedup vs baseline: 22.2609x; 22.2609x over previous
import functools

import jax
import jax.numpy as jnp
from jax import lax
from jax.experimental import pallas as pl
from jax.experimental.pallas import tpu as pltpu
from jax.experimental.pallas import tpu_sc as plsc

VOCAB = 100000
N_HIDDEN = 128
B = 16384
F = 26
TOT = B * F
NW = 32
CHUNK = TOT // NW
LANES = 16

VB = 12800
TC_GRID = (VOCAB + VB - 1) // VB


def _project_body(e_ref, w_ref, b_ref, o_ref):
    e = e_ref[...]
    w = w_ref[...]
    o_ref[...] = (
        lax.dot_general(w, e, (((1,), (1,)), ((), ())),
                        preferred_element_type=jnp.float32)
        + b_ref[0, 0]
    )


def _project_table(emb, W, b2d):
    return pl.pallas_call(
        _project_body,
        grid=(TC_GRID,),
        in_specs=[
            pl.BlockSpec((VB, N_HIDDEN), lambda i: (i, 0)),
            pl.BlockSpec((1, N_HIDDEN), lambda i: (0, 0)),
            pl.BlockSpec((1, 1), lambda i: (0, 0)),
        ],
        out_specs=pl.BlockSpec((1, VB), lambda i: (0, i)),
        out_shape=jax.ShapeDtypeStruct((1, VOCAB), jnp.float32),
    )(emb, W, b2d)


def _sc_gather_body(v_hbm, idx_hbm, out_hbm, v_v, idx_v, out_v):
    wid = lax.axis_index("s") * 2 + lax.axis_index("c")
    base = wid * CHUNK
    pltpu.sync_copy(v_hbm, v_v)
    pltpu.sync_copy(idx_hbm.at[pl.ds(base, CHUNK)], idx_v)

    def body(i, carry):
        idx = idx_v[pl.ds(i * LANES, LANES)]
        out_v[pl.ds(i * LANES, LANES)] = plsc.load_gather(v_v, [idx])
        return carry

    lax.fori_loop(0, CHUNK // LANES, body, 0)
    pltpu.sync_copy(out_v, out_hbm.at[pl.ds(base, CHUNK)])


@functools.cache
def _sc_gather():
    mesh = plsc.VectorSubcoreMesh(core_axis_name="c", subcore_axis_name="s")
    return pl.kernel(
        _sc_gather_body,
        out_type=jax.ShapeDtypeStruct((TOT,), jnp.float32),
        mesh=mesh,
        scratch_types=[
            pltpu.VMEM((VOCAB,), jnp.float32),
            pltpu.VMEM((CHUNK,), jnp.int32),
            pltpu.VMEM((CHUNK,), jnp.float32),
        ],
        compiler_params=pltpu.CompilerParams(needs_layout_passes=False),
    )


def kernel(x, emb, W, b):
    v = _project_table(emb, W, b.reshape(1, 1)).reshape(VOCAB)
    idx = x.reshape(TOT).astype(jnp.int32)
    out = _sc_gather()(v, idx)
    return out.reshape(B, F, 1)


# speedup vs baseline: 23.5586x; 1.0583x over previous
import functools

import jax
import jax.numpy as jnp
from jax import lax
from jax.experimental import pallas as pl
from jax.experimental.pallas import tpu as pltpu
from jax.experimental.pallas import tpu_sc as plsc

VOCAB = 100000
N_HIDDEN = 128
B = 16384
F = 26
TOT = B * F
NW = 32
CHUNK = TOT // NW
LANES = 16

VB = 12800
TC_GRID = (VOCAB + VB - 1) // VB


def _project_body(e_ref, w_ref, b_ref, o_ref):
    e = e_ref[...]
    w = w_ref[...]
    o_ref[...] = (
        lax.dot_general(w, e, (((1,), (1,)), ((), ())),
                        preferred_element_type=jnp.float32)
        + b_ref[0, 0]
    )


def _project_table(emb, W, b2d):
    return pl.pallas_call(
        _project_body,
        grid=(TC_GRID,),
        in_specs=[
            pl.BlockSpec((VB, N_HIDDEN), lambda i: (i, 0)),
            pl.BlockSpec((1, N_HIDDEN), lambda i: (0, 0)),
            pl.BlockSpec((1, 1), lambda i: (0, 0)),
        ],
        out_specs=pl.BlockSpec((1, VB), lambda i: (0, i)),
        out_shape=jax.ShapeDtypeStruct((1, VOCAB), jnp.float32),
    )(emb, W, b2d)


def _sc_gather_body(v_hbm, idx_hbm, out_hbm, v_v, idx_v, out_v, sem_v, sem_i):
    wid = lax.axis_index("s") * 2 + lax.axis_index("c")
    base = wid * CHUNK
    cp_v = pltpu.async_copy(v_hbm, v_v, sem_v)
    cp_i = pltpu.async_copy(idx_hbm.at[pl.ds(base, CHUNK)], idx_v, sem_i)
    cp_v.wait()
    cp_i.wait()

    @plsc.parallel_loop(0, CHUNK, step=LANES, unroll=8)
    def body(i):
        idx = idx_v[pl.ds(i, LANES)]
        out_v[pl.ds(i, LANES)] = plsc.load_gather(v_v, [idx])

    pltpu.sync_copy(out_v, out_hbm.at[pl.ds(base, CHUNK)])


@functools.cache
def _sc_gather():
    mesh = plsc.VectorSubcoreMesh(core_axis_name="c", subcore_axis_name="s")
    return pl.kernel(
        _sc_gather_body,
        out_type=jax.ShapeDtypeStruct((TOT,), jnp.float32),
        mesh=mesh,
        scratch_types=[
            pltpu.VMEM((VOCAB,), jnp.float32),
            pltpu.VMEM((CHUNK,), jnp.int32),
            pltpu.VMEM((CHUNK,), jnp.float32),
            pltpu.SemaphoreType.DMA,
            pltpu.SemaphoreType.DMA,
        ],
        compiler_params=pltpu.CompilerParams(needs_layout_passes=False),
    )


def kernel(x, emb, W, b):
    v = _project_table(emb, W, b.reshape(1, 1)).reshape(VOCAB)
    idx = x.reshape(TOT).astype(jnp.int32)
    out = _sc_gather()(v, idx)
    return out.reshape(B, F, 1)
